# baseline (speedup 1.0000x reference)
import functools

import jax
import jax.numpy as jnp
from jax import lax
from jax.experimental import pallas as pl
from jax.experimental.pallas import tpu as pltpu
from jax.experimental.pallas import tpu_sc as plsc

_HIST = 50
_BATCH = 4096
_D = 64
_NC = 2
_NS = 16
_NW = _NC * _NS
_BLK = _BATCH // _NW
_SCALE = 8.0

_mesh = plsc.VectorSubcoreMesh(
    core_axis_name="c", subcore_axis_name="s", num_cores=_NC, num_subcores=_NS
)


@functools.partial(
    pl.kernel,
    out_type=jax.ShapeDtypeStruct((_HIST, _D // 8, _NW, 8, _BLK), jnp.float32),
    mesh=_mesh,
    scratch_types=[
        pltpu.VMEM((_BLK,), jnp.int32),
        pltpu.VMEM((_BLK,), jnp.int32),
        pltpu.VMEM((_BLK,), jnp.int32),
        pltpu.VMEM((_BLK,), jnp.int32),
        pltpu.VMEM((_BLK, 2 * _D), jnp.float32),
        pltpu.VMEM((_BLK, 2 * _D), jnp.float32),
        pltpu.VMEM((_D, _BLK), jnp.float32),
        pltpu.VMEM((_D, _BLK), jnp.float32),
        pltpu.SemaphoreType.DMA,
        pltpu.SemaphoreType.DMA,
        pltpu.SemaphoreType.DMA,
        pltpu.SemaphoreType.DMA,
    ],
    compiler_params=pltpu.CompilerParams(
        use_tc_tiling_on_sc=True, needs_layout_passes=False
    ),
)
def _emb_lookup(tok_hbm, table_hbm, out_hbm,
                tokr0, tokr1, idx0, idx1, rows0, rows1, stg0, stg1,
                gsem0, gsem1, ssem0, ssem1):
    wid = lax.axis_index("s") * _NC + lax.axis_index("c")
    tokrs = (tokr0, tokr1)
    idxs = (idx0, idx1)
    rows = (rows0, rows1)
    stgs = (stg0, stg1)
    gsems = (gsem0, gsem1)
    ssems = (ssem0, ssem1)

    def prep_and_issue(b, h):
        pltpu.sync_copy(tok_hbm.at[h, wid], tokrs[b])

        def halve(k, carry):
            sl = pl.ds(16 * k, 16)
            idxs[b][sl] = lax.shift_right_logical(tokrs[b][sl], 1)
            return carry

        lax.fori_loop(0, _BLK // 16, halve, 0)
        pltpu.async_copy(table_hbm.at[idxs[b]], rows[b], gsems[b])

    def transform(b):
        buf = rows[b]
        stg = stgs[b]
        lane = lax.iota(jnp.int32, 16)

        def grp_body(k, carry):
            sl = pl.ds(16 * k, 16)
            row_idx = lane + 16 * k
            par = (tokrs[b][sl] & 1) * _D

            def f_body(f, carry2):
                v = plsc.load_gather(buf, [row_idx, par + f])
                stg[f, sl] = v * _SCALE
                return carry2

            lax.fori_loop(0, _D, f_body, 0)
            return carry

        lax.fori_loop(0, _BLK // 16, grp_body, 0)

    def wait_gather(b):
        pltpu.make_async_copy(table_hbm.at[idxs[b]], rows[b], gsems[b]).wait()

    def wait_out(b, h):
        for g in range(8):
            pltpu.make_async_copy(
                stgs[b].at[pl.ds(g * 8, 8)], out_hbm.at[h, g, wid], ssems[b]
            ).wait()

    def unit(b, h, g_iter):
        @pl.when(h + 1 < _HIST)
        def _():
            prep_and_issue(1 - b, h + 1)

        wait_gather(b)

        @pl.when(g_iter > 0)
        def _():
            wait_out(b, h)

        transform(b)
        for g in range(8):
            pltpu.async_copy(
                stgs[b].at[pl.ds(g * 8, 8)], out_hbm.at[h, g, wid], ssems[b]
            )

    prep_and_issue(0, 0)

    def body(g_iter, carry):
        unit(0, 2 * g_iter, g_iter)
        unit(1, 2 * g_iter + 1, g_iter)
        return carry

    lax.fori_loop(0, _HIST // 2, body, 0)

    wait_out(0, 0)
    wait_out(1, 0)


def kernel(b_tokens, table):
    tok3 = b_tokens.T.reshape(_HIST, _NW, _BLK).astype(jnp.int32)
    table2 = table.reshape(500000, 2 * _D)
    out5 = _emb_lookup(tok3, table2)
    return out5.transpose(2, 4, 0, 1, 3).reshape(_BATCH, _HIST, _D)
